# baseline (speedup 1.0000x reference)
import functools

import jax
import jax.numpy as jnp
from jax import lax
from jax.experimental import pallas as pl
from jax.experimental.pallas import tpu as pltpu
from jax.experimental.pallas import tpu_sc as plsc

N_NODES = 10000
D = 128
NC = 2
NS = 16
NW = NC * NS
CHUNK = 128
N_PAD = 10240
RPT = N_PAD // NS
DEGW = 128

def _mesh():
    return plsc.VectorSubcoreMesh(core_axis_name="c", subcore_axis_name="s",
                                  num_cores=NC, num_subcores=NS)


_FIRE = 8
_C0_FRAC = 0.65


def _deg_hist(dst3, zeros8, ones8, epw):
    epc = epw // CHUNK

    @functools.partial(
        pl.kernel,
        mesh=_mesh(),
        out_type=jax.ShapeDtypeStruct((NC, N_PAD, DEGW), jnp.float32),
        scratch_types=[
            pltpu.VMEM((epc, CHUNK), jnp.int32),
            pltpu.VMEM((CHUNK, DEGW), jnp.float32),
            pltpu.VMEM_SHARED((N_PAD, DEGW), jnp.float32),
            pltpu.SemaphoreType.DMA,
        ],
    )
    def k(dst_hbm, zeros_hbm, ones_hbm, out_hbm, idx_v, ones_v, acc_sh, sem):
        cid = lax.axis_index("c")
        sid = lax.axis_index("s")
        wid = cid * NS + sid
        pltpu.sync_copy(zeros_hbm.at[pl.ds(sid * RPT, RPT)],
                        acc_sh.at[pl.ds(sid * RPT, RPT)])
        pltpu.sync_copy(ones_hbm, ones_v)
        pltpu.sync_copy(dst_hbm.at[wid], idx_v)
        plsc.subcore_barrier()

        @pl.loop(0, epc, step=_FIRE)
        def body(g):
            for b in range(_FIRE):
                pltpu.async_copy(ones_v, acc_sh.at[idx_v.at[g + b]], sem,
                                 add=True)
            for b in range(_FIRE):
                pltpu.make_async_copy(ones_v, acc_sh.at[idx_v.at[g + b]],
                                      sem).wait()

        plsc.subcore_barrier()
        pltpu.sync_copy(acc_sh.at[pl.ds(sid * RPT, RPT)],
                        out_hbm.at[cid, pl.ds(sid * RPT, RPT)])

    return k(dst3, zeros8, ones8)


def _aggregate(y, sd3, zeros128, k0, k1):
    halfmax = max(k0, k1) // 2

    @functools.partial(
        pl.kernel,
        mesh=_mesh(),
        out_type=jax.ShapeDtypeStruct((NC, N_PAD, D), jnp.float32),
        scratch_types=[
            pltpu.VMEM((halfmax, 2, CHUNK), jnp.int32),
            pltpu.VMEM((CHUNK, D), jnp.float32),
            pltpu.VMEM((CHUNK, D), jnp.float32),
            pltpu.VMEM_SHARED((N_PAD, D), jnp.float32),
            pltpu.SemaphoreType.DMA,
            pltpu.SemaphoreType.DMA,
        ],
    )
    def k(y_hbm, sd_hbm, zeros_hbm, out_hbm,
          idx_v, rows0_v, rows1_v, acc_sh, gsem0, gsem1):
        cid = lax.axis_index("c")
        sid = lax.axis_index("s")
        pltpu.sync_copy(zeros_hbm.at[pl.ds(sid * RPT, RPT)],
                        acc_sh.at[pl.ds(sid * RPT, RPT)])
        plsc.subcore_barrier()

        rows = (rows0_v, rows1_v)
        gsems = (gsem0, gsem1)

        def fire(m, b):
            pltpu.async_copy(y_hbm.at[idx_v.at[m, 0]], rows[b], gsems[b])

        def wait(b):
            pltpu.make_async_copy(y_hbm.at[pl.ds(0, CHUNK)], rows[b],
                                  gsems[b]).wait()

        def scatter(m, b):
            pltpu.sync_copy(rows[b], acc_sh.at[idx_v.at[m, 1]], add=True)

        def run(kc, base):
            half = kc // 2
            for h in range(2):
                pltpu.sync_copy(sd_hbm.at[pl.ds(base + h * half, half)],
                                idx_v.at[pl.ds(0, half)])
                fire(0, 0)

                @pl.loop(0, half, step=2)
                def body(g):
                    fire(g + 1, 1)
                    wait(0)
                    scatter(g, 0)

                    @pl.when(g + 2 < half)
                    def _():
                        fire(g + 2, 0)

                    wait(1)
                    scatter(g + 1, 1)

        @pl.when(cid == 0)
        def _():
            run(k0, sid * k0)

        @pl.when(cid == 1)
        def _():
            run(k1, NS * k0 + sid * k1)

        plsc.subcore_barrier()
        pltpu.sync_copy(acc_sh.at[pl.ds(sid * RPT, RPT)],
                        out_hbm.at[cid, pl.ds(sid * RPT, RPT)])

    return k(y, sd3, zeros128)


_BR = 1024


def _dis_of(degp_ref):
    deg = degp_ref[0, :, 0] + degp_ref[1, :, 0] + 1.0
    return lax.rsqrt(deg)


def _tc1a_body(x_ref, w_ref, xw_ref):
    xw_ref[...] = jnp.dot(x_ref[...], w_ref[...],
                          preferred_element_type=jnp.float32)


def _tc1b_body(xw_ref, degp_ref, y_ref):
    dis = _dis_of(degp_ref)
    y_ref[...] = xw_ref[...] * dis[:, None]


def _tc2_body(s_ref, y1_ref, degp_ref, b_ref, w_ref, y2_ref):
    dis = _dis_of(degp_ref)
    agg = (s_ref[0] + s_ref[1] + y1_ref[...]) * dis[:, None] + b_ref[...]
    h = jnp.maximum(agg, 0.0)
    y2_ref[...] = jnp.dot(h, w_ref[...],
                          preferred_element_type=jnp.float32) * dis[:, None]


def _tc3_body(s_ref, y2_ref, degp_ref, b_ref, out_ref):
    dis = _dis_of(degp_ref)
    out_ref[...] = (s_ref[0] + s_ref[1] + y2_ref[...]) * dis[:, None] + b_ref[...]


def _row_specs():
    s_spec = pl.BlockSpec((2, _BR, D), lambda i: (0, i, 0))
    r_spec = pl.BlockSpec((_BR, D), lambda i: (i, 0))
    degp_spec = pl.BlockSpec((2, _BR, DEGW), lambda i: (0, i, 0))
    b_spec = pl.BlockSpec((1, D), lambda i: (0, 0))
    w_spec = pl.BlockSpec((D, D), lambda i: (0, 0))
    return s_spec, r_spec, degp_spec, b_spec, w_spec


def kernel(x, edge_index, W1, b1, W2, b2):
    n, d_in = x.shape
    e = edge_index.shape[1]
    grid = (N_PAD // _BR,)

    ei = edge_index.astype(jnp.int32)
    unit = NW * CHUNK * 2 * _FIRE
    epw = ((e + unit - 1) // unit) * CHUNK * 2 * _FIRE
    epc = epw // CHUNK
    e_pad = epw * NW
    pad = e_pad - e
    src_pad = jnp.concatenate([ei[0], jnp.full((pad,), n, jnp.int32)])
    dst_pad = jnp.concatenate([ei[1], jnp.full((pad,), n, jnp.int32)])
    dst3 = dst_pad.reshape(NW, epc, CHUNK)
    sd3 = jnp.stack(
        [src_pad.reshape(-1, CHUNK), dst_pad.reshape(-1, CHUNK)], axis=1)
    tot = e_pad // (NS * CHUNK)
    k0 = max(2, (int(tot * _C0_FRAC) // 2) * 2)
    k1 = tot - k0
    x_pad = jnp.pad(x, ((0, N_PAD - n), (0, 0)))
    zeros8 = jnp.zeros((N_PAD, DEGW), jnp.float32)
    ones8 = jnp.ones((CHUNK, DEGW), jnp.float32)
    zeros128 = jnp.zeros((N_PAD, D), jnp.float32)
    b1r = b1.reshape(1, D)
    b2r = b2.reshape(1, D)

    degp = _deg_hist(dst3, zeros8, ones8, epw)

    s_spec, r_spec, degp_spec, b_spec, w_spec = _row_specs()

    xw1 = pl.pallas_call(
        _tc1a_body,
        grid=grid,
        in_specs=[r_spec, w_spec],
        out_specs=r_spec,
        out_shape=jax.ShapeDtypeStruct((N_PAD, D), jnp.float32),
    )(x_pad, W1)

    y1 = pl.pallas_call(
        _tc1b_body,
        grid=grid,
        in_specs=[r_spec, degp_spec],
        out_specs=r_spec,
        out_shape=jax.ShapeDtypeStruct((N_PAD, D), jnp.float32),
    )(xw1, degp)

    s1 = _aggregate(y1, sd3, zeros128, k0, k1)

    y2 = pl.pallas_call(
        _tc2_body,
        grid=grid,
        in_specs=[s_spec, r_spec, degp_spec, b_spec, w_spec],
        out_specs=r_spec,
        out_shape=jax.ShapeDtypeStruct((N_PAD, D), jnp.float32),
    )(s1, y1, degp, b1r, W2)

    s2 = _aggregate(y2, sd3, zeros128, k0, k1)

    out = pl.pallas_call(
        _tc3_body,
        grid=grid,
        in_specs=[s_spec, r_spec, degp_spec, b_spec],
        out_specs=r_spec,
        out_shape=jax.ShapeDtypeStruct((N_PAD, D), jnp.float32),
    )(s2, y2, degp, b2r)

    return out[:n]


# speedup vs baseline: 1.0008x; 1.0008x over previous
import functools

import jax
import jax.numpy as jnp
from jax import lax
from jax.experimental import pallas as pl
from jax.experimental.pallas import tpu as pltpu
from jax.experimental.pallas import tpu_sc as plsc

N_NODES = 10000
D = 128
NC = 2
NS = 16
NW = NC * NS
CHUNK = 128
N_PAD = 10240
RPT = N_PAD // NS
DEGW = 128

def _mesh():
    return plsc.VectorSubcoreMesh(core_axis_name="c", subcore_axis_name="s",
                                  num_cores=NC, num_subcores=NS)


_FIRE = 8
_C0_FRAC = 0.75


def _deg_hist(dst3, zeros8, ones8, epw):
    epc = epw // CHUNK

    @functools.partial(
        pl.kernel,
        mesh=_mesh(),
        out_type=jax.ShapeDtypeStruct((NC, N_PAD, DEGW), jnp.float32),
        scratch_types=[
            pltpu.VMEM((epc, CHUNK), jnp.int32),
            pltpu.VMEM((CHUNK, DEGW), jnp.float32),
            pltpu.VMEM_SHARED((N_PAD, DEGW), jnp.float32),
            pltpu.SemaphoreType.DMA,
        ],
    )
    def k(dst_hbm, zeros_hbm, ones_hbm, out_hbm, idx_v, ones_v, acc_sh, sem):
        cid = lax.axis_index("c")
        sid = lax.axis_index("s")
        wid = cid * NS + sid
        pltpu.sync_copy(zeros_hbm.at[pl.ds(sid * RPT, RPT)],
                        acc_sh.at[pl.ds(sid * RPT, RPT)])
        pltpu.sync_copy(ones_hbm, ones_v)
        pltpu.sync_copy(dst_hbm.at[wid], idx_v)
        plsc.subcore_barrier()

        @pl.loop(0, epc, step=_FIRE)
        def body(g):
            for b in range(_FIRE):
                pltpu.async_copy(ones_v, acc_sh.at[idx_v.at[g + b]], sem,
                                 add=True)
            for b in range(_FIRE):
                pltpu.make_async_copy(ones_v, acc_sh.at[idx_v.at[g + b]],
                                      sem).wait()

        plsc.subcore_barrier()
        pltpu.sync_copy(acc_sh.at[pl.ds(sid * RPT, RPT)],
                        out_hbm.at[cid, pl.ds(sid * RPT, RPT)])

    return k(dst3, zeros8, ones8)


def _aggregate(y, sd3, zeros128, k0, k1):
    halfmax = max(k0, k1) // 2

    @functools.partial(
        pl.kernel,
        mesh=_mesh(),
        out_type=jax.ShapeDtypeStruct((NC, N_PAD, D), jnp.float32),
        scratch_types=[
            pltpu.VMEM((halfmax, 2, CHUNK), jnp.int32),
            pltpu.VMEM((CHUNK, D), jnp.float32),
            pltpu.VMEM((CHUNK, D), jnp.float32),
            pltpu.VMEM_SHARED((N_PAD, D), jnp.float32),
            pltpu.SemaphoreType.DMA,
            pltpu.SemaphoreType.DMA,
        ],
    )
    def k(y_hbm, sd_hbm, zeros_hbm, out_hbm,
          idx_v, rows0_v, rows1_v, acc_sh, gsem0, gsem1):
        cid = lax.axis_index("c")
        sid = lax.axis_index("s")
        pltpu.sync_copy(zeros_hbm.at[pl.ds(sid * RPT, RPT)],
                        acc_sh.at[pl.ds(sid * RPT, RPT)])
        plsc.subcore_barrier()

        rows = (rows0_v, rows1_v)
        gsems = (gsem0, gsem1)

        def fire(m, b):
            pltpu.async_copy(y_hbm.at[idx_v.at[m, 0]], rows[b], gsems[b])

        def wait(b):
            pltpu.make_async_copy(y_hbm.at[pl.ds(0, CHUNK)], rows[b],
                                  gsems[b]).wait()

        def scatter(m, b):
            pltpu.sync_copy(rows[b], acc_sh.at[idx_v.at[m, 1]], add=True)

        def run(kc, base):
            half = kc // 2
            for h in range(2):
                pltpu.sync_copy(sd_hbm.at[pl.ds(base + h * half, half)],
                                idx_v.at[pl.ds(0, half)])
                fire(0, 0)

                @pl.loop(0, half, step=2)
                def body(g):
                    fire(g + 1, 1)
                    wait(0)
                    scatter(g, 0)

                    @pl.when(g + 2 < half)
                    def _():
                        fire(g + 2, 0)

                    wait(1)
                    scatter(g + 1, 1)

        @pl.when(cid == 0)
        def _():
            run(k0, sid * k0)

        @pl.when(cid == 1)
        def _():
            run(k1, NS * k0 + sid * k1)

        plsc.subcore_barrier()
        pltpu.sync_copy(acc_sh.at[pl.ds(sid * RPT, RPT)],
                        out_hbm.at[cid, pl.ds(sid * RPT, RPT)])

    return k(y, sd3, zeros128)


_BR = 1024


def _dis_of(degp_ref):
    deg = degp_ref[0, :, 0] + degp_ref[1, :, 0] + 1.0
    return lax.rsqrt(deg)


def _tc1a_body(x_ref, w_ref, xw_ref):
    xw_ref[...] = jnp.dot(x_ref[...], w_ref[...],
                          preferred_element_type=jnp.float32)


def _tc1b_body(xw_ref, degp_ref, y_ref):
    dis = _dis_of(degp_ref)
    y_ref[...] = xw_ref[...] * dis[:, None]


def _tc2_body(s_ref, y1_ref, degp_ref, b_ref, w_ref, y2_ref):
    dis = _dis_of(degp_ref)
    agg = (s_ref[0] + s_ref[1] + y1_ref[...]) * dis[:, None] + b_ref[...]
    h = jnp.maximum(agg, 0.0)
    y2_ref[...] = jnp.dot(h, w_ref[...],
                          preferred_element_type=jnp.float32) * dis[:, None]


def _tc3_body(s_ref, y2_ref, degp_ref, b_ref, out_ref):
    dis = _dis_of(degp_ref)
    out_ref[...] = (s_ref[0] + s_ref[1] + y2_ref[...]) * dis[:, None] + b_ref[...]


def _row_specs():
    s_spec = pl.BlockSpec((2, _BR, D), lambda i: (0, i, 0))
    r_spec = pl.BlockSpec((_BR, D), lambda i: (i, 0))
    degp_spec = pl.BlockSpec((2, _BR, DEGW), lambda i: (0, i, 0))
    b_spec = pl.BlockSpec((1, D), lambda i: (0, 0))
    w_spec = pl.BlockSpec((D, D), lambda i: (0, 0))
    return s_spec, r_spec, degp_spec, b_spec, w_spec


def kernel(x, edge_index, W1, b1, W2, b2):
    n, d_in = x.shape
    e = edge_index.shape[1]
    grid = (N_PAD // _BR,)

    ei = edge_index.astype(jnp.int32)
    unit = NW * CHUNK * 2 * _FIRE
    epw = ((e + unit - 1) // unit) * CHUNK * 2 * _FIRE
    epc = epw // CHUNK
    e_pad = epw * NW
    pad = e_pad - e
    src_pad = jnp.concatenate([ei[0], jnp.full((pad,), n, jnp.int32)])
    dst_pad = jnp.concatenate([ei[1], jnp.full((pad,), n, jnp.int32)])
    dst3 = dst_pad.reshape(NW, epc, CHUNK)
    sd3 = jnp.stack(
        [src_pad.reshape(-1, CHUNK), dst_pad.reshape(-1, CHUNK)], axis=1)
    tot = e_pad // (NS * CHUNK)
    k0 = max(2, (int(tot * _C0_FRAC) // 2) * 2)
    k1 = tot - k0
    x_pad = jnp.pad(x, ((0, N_PAD - n), (0, 0)))
    zeros8 = jnp.zeros((N_PAD, DEGW), jnp.float32)
    ones8 = jnp.ones((CHUNK, DEGW), jnp.float32)
    zeros128 = jnp.zeros((N_PAD, D), jnp.float32)
    b1r = b1.reshape(1, D)
    b2r = b2.reshape(1, D)

    degp = _deg_hist(dst3, zeros8, ones8, epw)

    s_spec, r_spec, degp_spec, b_spec, w_spec = _row_specs()

    xw1 = pl.pallas_call(
        _tc1a_body,
        grid=grid,
        in_specs=[r_spec, w_spec],
        out_specs=r_spec,
        out_shape=jax.ShapeDtypeStruct((N_PAD, D), jnp.float32),
    )(x_pad, W1)

    y1 = pl.pallas_call(
        _tc1b_body,
        grid=grid,
        in_specs=[r_spec, degp_spec],
        out_specs=r_spec,
        out_shape=jax.ShapeDtypeStruct((N_PAD, D), jnp.float32),
    )(xw1, degp)

    s1 = _aggregate(y1, sd3, zeros128, k0, k1)

    y2 = pl.pallas_call(
        _tc2_body,
        grid=grid,
        in_specs=[s_spec, r_spec, degp_spec, b_spec, w_spec],
        out_specs=r_spec,
        out_shape=jax.ShapeDtypeStruct((N_PAD, D), jnp.float32),
    )(s1, y1, degp, b1r, W2)

    s2 = _aggregate(y2, sd3, zeros128, k0, k1)

    out = pl.pallas_call(
        _tc3_body,
        grid=grid,
        in_specs=[s_spec, r_spec, degp_spec, b_spec],
        out_specs=r_spec,
        out_shape=jax.ShapeDtypeStruct((N_PAD, D), jnp.float32),
    )(s2, y2, degp, b2r)

    return out[:n]
